# baseline (speedup 1.0000x reference)
import functools
import jax
import jax.numpy as jnp
from jax import lax
from jax.experimental import pallas as pl
from jax.experimental.pallas import tpu as pltpu
from jax.experimental.pallas import tpu_sc as plsc

_B, _T = 16, 4096
_NW = 32
_CHUNK = (_B * _T) // _NW


def _sc_pos_count(lab_hbm, lenw_hbm, out_hbm, lab_v, len_v, acc_v):
    wid = lax.axis_index("s") * 2 + lax.axis_index("c")
    base = wid * _CHUNK
    pltpu.sync_copy(lab_hbm.at[pl.ds(base, _CHUNK)], lab_v)
    pltpu.sync_copy(lenw_hbm.at[wid], len_v)

    iota16 = lax.iota(jnp.int32, 16)
    col_base = (wid % 2) * _CHUNK
    row_len = len_v[...]

    acc = jnp.zeros((16,), jnp.int32)
    for j in range(_CHUNK // 16):
        lab = lab_v[pl.ds(j * 16, 16)]
        col = col_base + j * 16 + iota16
        m = (lab == 1) & (col < row_len)
        acc = acc + jnp.where(m, 1, 0)
    acc_v[...] = acc
    pltpu.sync_copy(acc_v, out_hbm.at[wid])


_sc_count = functools.partial(
    pl.kernel,
    mesh=plsc.VectorSubcoreMesh(core_axis_name="c", subcore_axis_name="s"),
    out_type=jax.ShapeDtypeStruct((_NW, 16), jnp.int32),
    scratch_types=[
        pltpu.VMEM((_CHUNK,), jnp.int32),
        pltpu.VMEM((16,), jnp.int32),
        pltpu.VMEM((16,), jnp.int32),
    ],
)(_sc_pos_count)


def _loss_kernel(a_ref, b_ref, lcls_ref, bnd_ref, lbnd_ref, lenc_ref,
                 lenb_ref, pc_ref, total_ref, spoof_ref, bdry_ref):
    col = jax.lax.broadcasted_iota(jnp.int32, (_B, _T), 1)

    a = a_ref[...]
    b = b_ref[...]
    m = jnp.maximum(a, b)
    lse = m + jnp.log(jnp.exp(a - m) + jnp.exp(b - m))
    sel = jnp.where(lcls_ref[...] == 0, a, b)
    ce = lse - sel
    cmask = (col < lenc_ref[...]).astype(jnp.float32)
    spoof = jnp.sum(ce * cmask) / (jnp.sum(cmask) + 1e-8)

    pred = bnd_ref[...]
    tgt = lbnd_ref[...].astype(jnp.float32)
    bmask = (col < lenb_ref[...]).astype(jnp.float32)
    selp = jnp.where(tgt == 1.0, pred, 1.0 - pred)
    loss = jnp.minimum(-jnp.log(selp), 100.0) * bmask
    tgt_m = tgt * bmask
    pos = (tgt_m == 1.0).astype(jnp.float32)
    pos_count = jnp.sum(pc_ref[...]).astype(jnp.float32)
    neg_count_all = jnp.float32(_B * _T) - pos_count
    k = jnp.minimum(neg_count_all, jnp.floor(pos_count * 5.0))
    pos_loss = jnp.sum(loss * pos)
    neg_vals = loss * (1.0 - pos)

    vbits = jax.lax.bitcast_convert_type(neg_vals, jnp.int32)
    k_i = k.astype(jnp.int32)

    def radix_round(lo, s, njs):
        t = jnp.int32(0)
        for j in range(1, njs + 1):
            mm = lo + (j << s)
            c = jnp.sum((vbits >= mm).astype(jnp.int32))
            t = t + (c >= k_i).astype(jnp.int32)
        return lo + t * (1 << s)

    lo = jnp.int32(0)
    for s in (29, 27, 25, 23, 21, 19, 17, 15, 13, 11, 9, 7, 5, 3, 1):
        lo = radix_round(lo, s, 3)
    lo = radix_round(lo, 0, 1)

    t = jax.lax.bitcast_convert_type(lo, jnp.float32)
    gt = vbits > lo
    cnt_gt = jnp.sum(gt.astype(jnp.float32))
    sum_gt = jnp.sum(jnp.where(gt, neg_vals, 0.0))
    neg_loss = jnp.where(k_i == 0, 0.0, sum_gt + (k - cnt_gt) * t)

    balanced = (pos_loss + neg_loss) / (pos_count + k + 1e-8)
    mean_loss = jnp.sum(loss) / jnp.float32(_B * _T)
    bdry = jnp.where(pos_count == 0.0, mean_loss, balanced)

    total_ref[...] = jnp.broadcast_to(spoof + 0.5 * bdry, (1, 1))
    spoof_ref[...] = jnp.broadcast_to(spoof, (1, 1))
    bdry_ref[...] = jnp.broadcast_to(bdry, (1, 1))


@jax.jit
def kernel(output, boundary, label_cls, label_boundary, len_cls, len_boundary):
    a = output[:, :, 0]
    b = output[:, :, 1]
    lenc = len_cls.reshape(_B, 1)
    lenb = len_boundary.reshape(_B, 1)
    lenw = jnp.broadcast_to(jnp.repeat(len_boundary, 2)[:, None], (_NW, 16))
    pc_parts = _sc_count(label_boundary.reshape(-1), lenw)
    total, spoof, bdry = pl.pallas_call(
        _loss_kernel,
        out_shape=[jax.ShapeDtypeStruct((1, 1), jnp.float32)] * 3,
    )(a, b, label_cls, boundary, label_boundary, lenc, lenb, pc_parts)
    return (total.reshape(()), spoof.reshape(()), bdry.reshape(()))


# speedup vs baseline: 2.3460x; 2.3460x over previous
import jax
import jax.numpy as jnp
from jax.experimental import pallas as pl

_B, _T = 16, 4096
_NCH = 4
_CW = _T // _NCH


def _loss_kernel(a_ref, b_ref, lcls_ref, bnd_ref, lbnd_ref, lenc_ref,
                 lenb_ref, total_ref, spoof_ref, bdry_ref):
    col = jax.lax.broadcasted_iota(jnp.int32, (_B, _T), 1)

    a = a_ref[...]
    b = b_ref[...]
    m = jnp.maximum(a, b)
    lse = m + jnp.log(jnp.exp(a - m) + jnp.exp(b - m))
    sel = jnp.where(lcls_ref[...] == 0, a, b)
    ce = lse - sel
    ccond = col < lenc_ref[...]
    csum = jnp.sum(lenc_ref[...]).astype(jnp.float32)
    spoof = jnp.sum(jnp.where(ccond, ce, 0.0)) / (csum + 1e-8)

    pred = bnd_ref[...]
    tgt = lbnd_ref[...]
    bcond = col < lenb_ref[...]
    selp = jnp.where(tgt == 1, pred, 1.0 - pred)
    rawloss = jnp.minimum(-jnp.log(selp), 100.0)
    posb = bcond & (tgt == 1)
    loss = jnp.where(bcond, rawloss, 0.0)
    pos_count = jnp.sum(jnp.where(posb, 1.0, 0.0))
    neg_count_all = jnp.float32(_B * _T) - pos_count
    k = jnp.minimum(neg_count_all, jnp.floor(pos_count * 5.0))
    pos_loss = jnp.sum(jnp.where(posb, rawloss, 0.0))
    neg_vals = jnp.where(posb, 0.0, loss)

    vbits = jax.lax.bitcast_convert_type(neg_vals, jnp.int32)
    k_i = k.astype(jnp.int32)

    def count_ge(mth):
        return jnp.sum((vbits >= mth).astype(jnp.int32))

    def radix_round(lo, s, njs):
        t = jnp.int32(0)
        for j in range(1, njs + 1):
            c = count_ge(lo + (j << s))
            t = t + (c >= k_i).astype(jnp.int32)
        return lo + t * (1 << s)

    lo = radix_round(jnp.int32(0), 29, 2)
    for s in (27, 25, 23, 21, 19, 17, 15, 13, 11, 9, 7, 5, 3, 1):
        lo = radix_round(lo, s, 3)
    lo = radix_round(lo, 0, 1)

    t = jax.lax.bitcast_convert_type(lo, jnp.float32)
    gt = vbits > lo
    cnt_gt = jnp.sum(jnp.where(gt, 1.0, 0.0))
    sum_gt = jnp.sum(jnp.where(gt, neg_vals, 0.0))
    neg_loss = jnp.where(k_i == 0, 0.0, sum_gt + (k - cnt_gt) * t)

    balanced = (pos_loss + neg_loss) / (pos_count + k + 1e-8)
    mean_loss = jnp.sum(loss) / jnp.float32(_B * _T)
    bdry = jnp.where(pos_count == 0.0, mean_loss, balanced)

    total_ref[...] = jnp.broadcast_to(spoof + 0.5 * bdry, (1, 1))
    spoof_ref[...] = jnp.broadcast_to(spoof, (1, 1))
    bdry_ref[...] = jnp.broadcast_to(bdry, (1, 1))


@jax.jit
def kernel(output, boundary, label_cls, label_boundary, len_cls, len_boundary):
    a = output[:, :, 0]
    b = output[:, :, 1]
    lenc = len_cls.reshape(_B, 1)
    lenb = len_boundary.reshape(_B, 1)
    total, spoof, bdry = pl.pallas_call(
        _loss_kernel,
        out_shape=[jax.ShapeDtypeStruct((1, 1), jnp.float32)] * 3,
    )(a, b, label_cls, boundary, label_boundary, lenc, lenb)
    return (total.reshape(()), spoof.reshape(()), bdry.reshape(()))
